# baseline (speedup 1.0000x reference)
import functools

import jax
import jax.numpy as jnp
from jax import lax
from jax.experimental import pallas as pl
from jax.experimental.pallas import tpu as pltpu
from jax.experimental.pallas import tpu_sc as plsc

N = 131072
D = 64
K = 1024
C = 8
NOISE_LOG_PROP = -5.0

NC = 2
NS = 16
NW = NC * NS
L = 16

SPW = N // NW
CHUNK = 64
GPC = CHUNK // L
NCHUNK = SPW // CHUNK
DCH = 8
FTS = D + 1

_SORT_NET = (
    (0, 1), (2, 3), (4, 5), (6, 7),
    (0, 2), (1, 3), (4, 6), (5, 7),
    (1, 2), (5, 6),
    (0, 4), (1, 5), (2, 6), (3, 7),
    (2, 4), (3, 5),
    (1, 2), (3, 4), (5, 6),
)


def _sc_body(feat_hbm, cand_hbm, meansT_hbm, lp_hbm,
             mparts_hbm, nuparts_hbm, newcand_hbm, mx_hbm, se_hbm, nll_hbm,
             means_v, qbuf, fbuf0, fbuf1, ft0, ft1, cbuf0, cbuf1,
             ncbuf0, ncbuf1, mxbuf0, mxbuf1, sebuf0, sebuf1,
             nllbuf0, nllbuf1, btab_v, b2_v, nu_v,
             insem0, insem1, outsem0, outsem1):
  wid = lax.axis_index("s") * NC + lax.axis_index("c")
  fbufs = (fbuf0, fbuf1)
  fts = (ft0, ft1)
  cbufs = (cbuf0, cbuf1)
  ncbufs = (ncbuf0, ncbuf1)
  mxbufs = (mxbuf0, mxbuf1)
  sebufs = (sebuf0, sebuf1)
  nllbufs = (nllbuf0, nllbuf1)
  insems = (insem0, insem1)
  outsems = (outsem0, outsem1)

  lane = lax.iota(jnp.int32, L)
  lane8 = lane * C
  laneft = lane * FTS
  zero = jnp.zeros((L,), jnp.float32)

  def in_descs(ch, p):
    base = wid * SPW + ch * CHUNK
    fd = pltpu.make_async_copy(
        feat_hbm.at[pl.ds(base * D, CHUNK * D)], fbufs[p], insems[p])
    cd = pltpu.make_async_copy(
        cand_hbm.at[pl.ds(base * C, CHUNK * C)], cbufs[p], insems[p])
    return fd, cd

  def start_in(ch, p):
    fd, cd = in_descs(ch, p)
    fd.start()
    cd.start()

  def wait_in(ch, p):
    fd, cd = in_descs(ch, p)
    fd.wait()
    cd.wait()

  def out_descs(ch, p):
    base = wid * SPW + ch * CHUNK
    return (
        pltpu.make_async_copy(
            ncbufs[p], newcand_hbm.at[pl.ds(base * C, CHUNK * C)],
            outsems[p]),
        pltpu.make_async_copy(
            mxbufs[p], mx_hbm.at[pl.ds(base, CHUNK)], outsems[p]),
        pltpu.make_async_copy(
            sebufs[p], se_hbm.at[pl.ds(base, CHUNK)], outsems[p]),
        pltpu.make_async_copy(
            nllbufs[p], nll_hbm.at[pl.ds(base, CHUNK)], outsems[p]),
    )

  def transpose_f(p):
    fbuf = fbufs[p]
    ft = fts[p]

    def tloop(s, _):
      so = s * D
      for q in range(D // L):
        v = fbuf[pl.ds(so + q * L, L)]
        plsc.store_scatter(ft, [laneft + (q * L * FTS + s)], v)
      return 0
    lax.fori_loop(0, CHUNK, tloop, 0, unroll=4)

  start_in(0, 0)
  start_in(1, 1)
  pltpu.sync_copy(meansT_hbm, means_v)
  pltpu.sync_copy(lp_hbm, btab_v)

  def b2_loop(kk, _):
    def dstep(d, a):
      v = means_v[pl.ds(d * K + kk * L, L)]
      return a + v * v
    nsq = lax.fori_loop(0, D, dstep, zero, unroll=8)
    b2_v[pl.ds(kk * L, L)] = btab_v[pl.ds(kk * L, L)] - 0.5 * nsq
    nu_v[pl.ds(kk * L, L)] = zero
    return 0
  lax.fori_loop(0, K // L, b2_loop, 0)

  def group_a(ch, g, ft, cbuf, ncbuf, mxbuf, sebuf, nllbuf):
    coff = g * (L * C)
    cands = [plsc.load_gather(cbuf, [lane8 + (coff + c)]) for c in range(C)]

    def dloop(dc, carry):
      dots = list(carry[:C])
      nacc = carry[C]
      d0 = dc * DCH
      fs = [ft[pl.ds((d0 + dd) * FTS + g * L, L)] for dd in range(DCH)]
      for dd in range(DCH):
        nacc = nacc + fs[dd] * fs[dd]
      for c in range(C):
        dot = dots[c]
        for dd in range(DCH):
          mu = plsc.load_gather(
              means_v.at[pl.ds((d0 + dd) * K, K)], [cands[c]])
          dot = dot + fs[dd] * mu
        dots[c] = dot
      return tuple(dots) + (nacc,)

    out = lax.fori_loop(0, D // DCH, dloop, tuple([zero] * C) + (zero,),
                        unroll=2)
    nacc = out[C]

    b2s = [plsc.load_gather(b2_v, [cands[c]]) for c in range(C)]
    lls = [b2s[c] + out[c] for c in range(C)]

    mx = jnp.full((L,), NOISE_LOG_PROP, jnp.float32)
    for c in range(C):
      mx = jnp.maximum(mx, lls[c])
    es = [jnp.exp(lls[c] - mx) for c in range(C)]
    se = jnp.exp(NOISE_LOG_PROP - mx)
    for c in range(C):
      se = se + es[c]
    r = 1.0 / se
    qs = [es[c] * r for c in range(C)]

    soff = g * L
    mxbuf[pl.ds(soff, L)] = mx - 0.5 * nacc
    sebuf[pl.ds(soff, L)] = se
    nllbuf[pl.ds(soff, L)] = NOISE_LOG_PROP - 0.5 * nacc

    goff = (ch * GPC + g) * (L * C)
    for c in range(C):
      plsc.addupdate_scatter(nu_v, [cands[c]], qs[c])
      qbuf[pl.ds(goff + c * L, L)] = qs[c]

    keys = list(lls)
    vals = list(cands)
    for (i, j) in _SORT_NET:
      m = keys[i] >= keys[j]
      ki = jnp.where(m, keys[i], keys[j])
      kj = jnp.where(m, keys[j], keys[i])
      vi = jnp.where(m, vals[i], vals[j])
      vj = jnp.where(m, vals[j], vals[i])
      keys[i], keys[j] = ki, kj
      vals[i], vals[j] = vi, vj
    for c in range(C):
      plsc.store_scatter(ncbuf, [lane8 + (coff + c)], vals[c])

  def chunk_a(i, _):
    for p in range(2):
      ch = 2 * i + p
      wait_in(ch, p)
      transpose_f(p)

      @pl.when(ch >= 2)
      def _():
        for dsc in out_descs(ch - 2, p):
          dsc.wait()

      def ga(g, _):
        group_a(ch, g, fts[p], cbufs[p], ncbufs[p],
                mxbufs[p], sebufs[p], nllbufs[p])
        return 0
      lax.fori_loop(0, GPC, ga, 0)

      for dsc in out_descs(ch, p):
        dsc.start()

      @pl.when(ch + 2 < NCHUNK)
      def _():
        start_in(ch + 2, p)
    return 0

  lax.fori_loop(0, NCHUNK // 2, chunk_a, 0)

  start_in(0, 0)
  start_in(1, 1)

  for p in range(2):
    for dsc in out_descs(NCHUNK - 2 + p, p):
      dsc.wait()

  def zero_m(i, _):
    means_v[pl.ds(i * L, L)] = zero
    return 0
  lax.fori_loop(0, (K * D) // L, zero_m, 0, unroll=8)

  def group_b(ch, g, ft, cbuf):
    coff = g * (L * C)
    cands = [plsc.load_gather(cbuf, [lane8 + (coff + c)]) for c in range(C)]
    goff = (ch * GPC + g) * (L * C)
    qs = [qbuf[pl.ds(goff + c * L, L)] for c in range(C)]

    def dloop(dc, _):
      d0 = dc * DCH
      fs = [ft[pl.ds((d0 + dd) * FTS + g * L, L)] for dd in range(DCH)]
      for c in range(C):
        for dd in range(DCH):
          plsc.addupdate_scatter(
              means_v.at[pl.ds((d0 + dd) * K, K)], [cands[c]],
              qs[c] * fs[dd])
      return 0

    lax.fori_loop(0, D // DCH, dloop, 0, unroll=2)

  def chunk_b(i, _):
    for p in range(2):
      ch = 2 * i + p
      wait_in(ch, p)
      transpose_f(p)

      def gb(g, _):
        group_b(ch, g, fts[p], cbufs[p])
        return 0
      lax.fori_loop(0, GPC, gb, 0)

      @pl.when(ch + 2 < NCHUNK)
      def _():
        start_in(ch + 2, p)
    return 0

  lax.fori_loop(0, NCHUNK // 2, chunk_b, 0)

  pltpu.sync_copy(means_v, mparts_hbm.at[pl.ds(wid * (K * D), K * D)])
  pltpu.sync_copy(nu_v, nuparts_hbm.at[pl.ds(wid * K, K)])


_sc_kernel = functools.partial(
    pl.kernel,
    out_type=[
        jax.ShapeDtypeStruct((NW * K * D,), jnp.float32),
        jax.ShapeDtypeStruct((NW * K,), jnp.float32),
        jax.ShapeDtypeStruct((N * C,), jnp.int32),
        jax.ShapeDtypeStruct((N,), jnp.float32),
        jax.ShapeDtypeStruct((N,), jnp.float32),
        jax.ShapeDtypeStruct((N,), jnp.float32),
    ],
    mesh=plsc.VectorSubcoreMesh(
        core_axis_name="c", subcore_axis_name="s",
        num_cores=NC, num_subcores=NS),
    compiler_params=pltpu.CompilerParams(needs_layout_passes=False),
    scratch_types=[
        pltpu.VMEM((K * D,), jnp.float32),
        pltpu.VMEM((SPW * C,), jnp.float32),
        pltpu.VMEM((CHUNK * D,), jnp.float32),
        pltpu.VMEM((CHUNK * D,), jnp.float32),
        pltpu.VMEM((D * FTS,), jnp.float32),
        pltpu.VMEM((D * FTS,), jnp.float32),
        pltpu.VMEM((CHUNK * C,), jnp.int32),
        pltpu.VMEM((CHUNK * C,), jnp.int32),
        pltpu.VMEM((CHUNK * C,), jnp.int32),
        pltpu.VMEM((CHUNK * C,), jnp.int32),
        pltpu.VMEM((CHUNK,), jnp.float32),
        pltpu.VMEM((CHUNK,), jnp.float32),
        pltpu.VMEM((CHUNK,), jnp.float32),
        pltpu.VMEM((CHUNK,), jnp.float32),
        pltpu.VMEM((CHUNK,), jnp.float32),
        pltpu.VMEM((CHUNK,), jnp.float32),
        pltpu.VMEM((K,), jnp.float32),
        pltpu.VMEM((K,), jnp.float32),
        pltpu.VMEM((K,), jnp.float32),
        pltpu.SemaphoreType.DMA,
        pltpu.SemaphoreType.DMA,
        pltpu.SemaphoreType.DMA,
        pltpu.SemaphoreType.DMA,
    ],
)(_sc_body)


def _tc_stats(nuparts_ref, mx_ref, se_ref, nll_ref, nu_ref, elbo_ref, nn_ref):
  nu = jnp.sum(nuparts_ref[...], axis=0, keepdims=True)
  nu_ref[...] = nu
  mx = mx_ref[...]
  se = se_ref[...]
  logz = mx + jnp.log(se)
  elbo = jnp.sum(logz) * (1.0 / N)
  qn = jnp.exp(nll_ref[...] - mx) / se
  nn = jnp.sum(qn)
  elbo_ref[...] = jnp.full((8, 128), elbo, jnp.float32)
  nn_ref[...] = jnp.full((8, 128), nn, jnp.float32)


def _tc_mreduce(mpart_ref, nu_ref, m_ref, acc_ref):
  i = pl.program_id(0)

  @pl.when(i == 0)
  def _():
    acc_ref[...] = jnp.zeros_like(acc_ref)

  acc_ref[...] += mpart_ref[0]

  @pl.when(i == NW - 1)
  def _():
    nub = lax.broadcast_in_dim(nu_ref[0], (D, K), (1,))
    m_ref[...] = (acc_ref[...] / jnp.clip(nub, 1.0, None)).T


def kernel(features, means, log_proportions, candidates):
  feat_flat = features.reshape(-1)
  cand_flat = candidates.reshape(-1)
  meansT_flat = means.T.reshape(-1)

  mparts, nuparts, newcand, mx, se, nll = _sc_kernel(
      feat_flat, cand_flat, meansT_flat, log_proportions)

  nu, elbo, nn = pl.pallas_call(
      _tc_stats,
      out_shape=[
          jax.ShapeDtypeStruct((1, K), jnp.float32),
          jax.ShapeDtypeStruct((8, 128), jnp.float32),
          jax.ShapeDtypeStruct((8, 128), jnp.float32),
      ],
  )(nuparts.reshape(NW, K),
    mx.reshape(N // 128, 128), se.reshape(N // 128, 128),
    nll.reshape(N // 128, 128))

  m = pl.pallas_call(
      _tc_mreduce,
      grid=(NW,),
      in_specs=[
          pl.BlockSpec((1, D, K), lambda i: (i, 0, 0)),
          pl.BlockSpec((1, K), lambda i: (0, 0)),
      ],
      out_specs=pl.BlockSpec((K, D), lambda i: (0, 0)),
      out_shape=jax.ShapeDtypeStruct((K, D), jnp.float32),
      scratch_shapes=[pltpu.VMEM((D, K), jnp.float32)],
  )(mparts.reshape(NW, D, K), nu)

  N_units = nu.reshape(K)
  noise_N = nn[0, 0]
  obs_elbo = elbo[0, 0]
  new_candidates = newcand.reshape(N, C)
  return m, N_units, noise_N, obs_elbo, new_candidates


# speedup vs baseline: 1.0191x; 1.0191x over previous
import functools

import jax
import jax.numpy as jnp
from jax import lax
from jax.experimental import pallas as pl
from jax.experimental.pallas import tpu as pltpu
from jax.experimental.pallas import tpu_sc as plsc

N = 131072
D = 64
K = 1024
C = 8
NOISE_LOG_PROP = -5.0

NC = 2
NS = 16
NW = NC * NS
L = 16

SPW = N // NW
CHUNK = 64
GPC = CHUNK // L
NCHUNK = SPW // CHUNK
DCH = 8
FTS = D + 1

_SORT_NET = (
    (0, 1), (2, 3), (4, 5), (6, 7),
    (0, 2), (1, 3), (4, 6), (5, 7),
    (1, 2), (5, 6),
    (0, 4), (1, 5), (2, 6), (3, 7),
    (2, 4), (3, 5),
    (1, 2), (3, 4), (5, 6),
)


def _sc_body(feat_hbm, cand_hbm, meansT_hbm, lp_hbm,
             mparts_hbm, nuparts_hbm, newcand_hbm, mx_hbm, se_hbm, nll_hbm,
             means_v, qbuf, fbuf0, fbuf1, ft0, ft1, cbuf0, cbuf1,
             ncbuf0, ncbuf1, mxbuf0, mxbuf1, sebuf0, sebuf1,
             nllbuf0, nllbuf1, btab_v, b2_v, nu_v,
             insem0, insem1, outsem0, outsem1):
  wid = lax.axis_index("s") * NC + lax.axis_index("c")
  fbufs = (fbuf0, fbuf1)
  fts = (ft0, ft1)
  cbufs = (cbuf0, cbuf1)
  ncbufs = (ncbuf0, ncbuf1)
  mxbufs = (mxbuf0, mxbuf1)
  sebufs = (sebuf0, sebuf1)
  nllbufs = (nllbuf0, nllbuf1)
  insems = (insem0, insem1)
  outsems = (outsem0, outsem1)

  lane = lax.iota(jnp.int32, L)
  lane8 = lane * C
  laneft = lane * FTS
  zero = jnp.zeros((L,), jnp.float32)

  def in_descs(ch, p):
    base = wid * SPW + ch * CHUNK
    fd = pltpu.make_async_copy(
        feat_hbm.at[pl.ds(base * D, CHUNK * D)], fbufs[p], insems[p])
    cd = pltpu.make_async_copy(
        cand_hbm.at[pl.ds(base * C, CHUNK * C)], cbufs[p], insems[p])
    return fd, cd

  def start_in(ch, p):
    fd, cd = in_descs(ch, p)
    fd.start()
    cd.start()

  def wait_in(ch, p):
    fd, cd = in_descs(ch, p)
    fd.wait()
    cd.wait()

  def out_descs(ch, p):
    base = wid * SPW + ch * CHUNK
    return (
        pltpu.make_async_copy(
            ncbufs[p], newcand_hbm.at[pl.ds(base * C, CHUNK * C)],
            outsems[p]),
        pltpu.make_async_copy(
            mxbufs[p], mx_hbm.at[pl.ds(base, CHUNK)], outsems[p]),
        pltpu.make_async_copy(
            sebufs[p], se_hbm.at[pl.ds(base, CHUNK)], outsems[p]),
        pltpu.make_async_copy(
            nllbufs[p], nll_hbm.at[pl.ds(base, CHUNK)], outsems[p]),
    )

  def transpose_f(p):
    fbuf = fbufs[p]
    ft = fts[p]

    def tloop(s, _):
      so = s * D
      for q in range(D // L):
        v = fbuf[pl.ds(so + q * L, L)]
        plsc.store_scatter(ft, [laneft + (q * L * FTS + s)], v)
      return 0
    lax.fori_loop(0, CHUNK, tloop, 0)

  start_in(0, 0)
  start_in(1, 1)
  pltpu.sync_copy(meansT_hbm, means_v)
  pltpu.sync_copy(lp_hbm, btab_v)

  def b2_loop(kk, _):
    def dstep(d, a):
      v = means_v[pl.ds(d * K + kk * L, L)]
      return a + v * v
    nsq = lax.fori_loop(0, D, dstep, zero)
    b2_v[pl.ds(kk * L, L)] = btab_v[pl.ds(kk * L, L)] - 0.5 * nsq
    nu_v[pl.ds(kk * L, L)] = zero
    return 0
  lax.fori_loop(0, K // L, b2_loop, 0)

  def group_a(ch, g, ft, cbuf, ncbuf, mxbuf, sebuf, nllbuf):
    coff = g * (L * C)
    cands = [plsc.load_gather(cbuf, [lane8 + (coff + c)]) for c in range(C)]

    def dloop(dc, carry):
      dots = list(carry[:C])
      nacc = carry[C]
      d0 = dc * DCH
      fs = [ft[pl.ds((d0 + dd) * FTS + g * L, L)] for dd in range(DCH)]
      for dd in range(DCH):
        nacc = nacc + fs[dd] * fs[dd]
      for c in range(C):
        dot = dots[c]
        for dd in range(DCH):
          mu = plsc.load_gather(
              means_v.at[pl.ds((d0 + dd) * K, K)], [cands[c]])
          dot = dot + fs[dd] * mu
        dots[c] = dot
      return tuple(dots) + (nacc,)

    out = lax.fori_loop(0, D // DCH, dloop, tuple([zero] * C) + (zero,))
    nacc = out[C]

    b2s = [plsc.load_gather(b2_v, [cands[c]]) for c in range(C)]
    lls = [b2s[c] + out[c] for c in range(C)]

    mx = jnp.full((L,), NOISE_LOG_PROP, jnp.float32)
    for c in range(C):
      mx = jnp.maximum(mx, lls[c])
    es = [jnp.exp(lls[c] - mx) for c in range(C)]
    se = jnp.exp(NOISE_LOG_PROP - mx)
    for c in range(C):
      se = se + es[c]
    r = 1.0 / se
    qs = [es[c] * r for c in range(C)]

    soff = g * L
    mxbuf[pl.ds(soff, L)] = mx - 0.5 * nacc
    sebuf[pl.ds(soff, L)] = se
    nllbuf[pl.ds(soff, L)] = NOISE_LOG_PROP - 0.5 * nacc

    goff = (ch * GPC + g) * (L * C)
    for c in range(C):
      plsc.addupdate_scatter(nu_v, [cands[c]], qs[c])
      qbuf[pl.ds(goff + c * L, L)] = qs[c]

    keys = list(lls)
    vals = list(cands)
    for (i, j) in _SORT_NET:
      m = keys[i] >= keys[j]
      ki = jnp.where(m, keys[i], keys[j])
      kj = jnp.where(m, keys[j], keys[i])
      vi = jnp.where(m, vals[i], vals[j])
      vj = jnp.where(m, vals[j], vals[i])
      keys[i], keys[j] = ki, kj
      vals[i], vals[j] = vi, vj
    for c in range(C):
      plsc.store_scatter(ncbuf, [lane8 + (coff + c)], vals[c])

  def chunk_a(i, _):
    for p in range(2):
      ch = 2 * i + p
      wait_in(ch, p)
      transpose_f(p)

      @pl.when(ch >= 2)
      def _():
        for dsc in out_descs(ch - 2, p):
          dsc.wait()

      def ga(g, _):
        group_a(ch, g, fts[p], cbufs[p], ncbufs[p],
                mxbufs[p], sebufs[p], nllbufs[p])
        return 0
      lax.fori_loop(0, GPC, ga, 0)

      for dsc in out_descs(ch, p):
        dsc.start()

      @pl.when(ch + 2 < NCHUNK)
      def _():
        start_in(ch + 2, p)
    return 0

  lax.fori_loop(0, NCHUNK // 2, chunk_a, 0)

  start_in(0, 0)
  start_in(1, 1)

  for p in range(2):
    for dsc in out_descs(NCHUNK - 2 + p, p):
      dsc.wait()

  def zero_m(i, _):
    means_v[pl.ds(i * L, L)] = zero
    return 0
  lax.fori_loop(0, (K * D) // L, zero_m, 0)

  def group_b(ch, g, ft, cbuf):
    coff = g * (L * C)
    cands = [plsc.load_gather(cbuf, [lane8 + (coff + c)]) for c in range(C)]
    goff = (ch * GPC + g) * (L * C)
    qs = [qbuf[pl.ds(goff + c * L, L)] for c in range(C)]

    def dloop(dc, _):
      d0 = dc * DCH
      fs = [ft[pl.ds((d0 + dd) * FTS + g * L, L)] for dd in range(DCH)]
      for c in range(C):
        for dd in range(DCH):
          plsc.addupdate_scatter(
              means_v.at[pl.ds((d0 + dd) * K, K)], [cands[c]],
              qs[c] * fs[dd])
      return 0

    lax.fori_loop(0, D // DCH, dloop, 0)

  def chunk_b(i, _):
    for p in range(2):
      ch = 2 * i + p
      wait_in(ch, p)
      transpose_f(p)

      def gb(g, _):
        group_b(ch, g, fts[p], cbufs[p])
        return 0
      lax.fori_loop(0, GPC, gb, 0)

      @pl.when(ch + 2 < NCHUNK)
      def _():
        start_in(ch + 2, p)
    return 0

  lax.fori_loop(0, NCHUNK // 2, chunk_b, 0)

  pltpu.sync_copy(means_v, mparts_hbm.at[pl.ds(wid * (K * D), K * D)])
  pltpu.sync_copy(nu_v, nuparts_hbm.at[pl.ds(wid * K, K)])


_sc_kernel = functools.partial(
    pl.kernel,
    out_type=[
        jax.ShapeDtypeStruct((NW * K * D,), jnp.float32),
        jax.ShapeDtypeStruct((NW * K,), jnp.float32),
        jax.ShapeDtypeStruct((N * C,), jnp.int32),
        jax.ShapeDtypeStruct((N,), jnp.float32),
        jax.ShapeDtypeStruct((N,), jnp.float32),
        jax.ShapeDtypeStruct((N,), jnp.float32),
    ],
    mesh=plsc.VectorSubcoreMesh(
        core_axis_name="c", subcore_axis_name="s",
        num_cores=NC, num_subcores=NS),
    compiler_params=pltpu.CompilerParams(needs_layout_passes=False),
    scratch_types=[
        pltpu.VMEM((K * D,), jnp.float32),
        pltpu.VMEM((SPW * C,), jnp.float32),
        pltpu.VMEM((CHUNK * D,), jnp.float32),
        pltpu.VMEM((CHUNK * D,), jnp.float32),
        pltpu.VMEM((D * FTS,), jnp.float32),
        pltpu.VMEM((D * FTS,), jnp.float32),
        pltpu.VMEM((CHUNK * C,), jnp.int32),
        pltpu.VMEM((CHUNK * C,), jnp.int32),
        pltpu.VMEM((CHUNK * C,), jnp.int32),
        pltpu.VMEM((CHUNK * C,), jnp.int32),
        pltpu.VMEM((CHUNK,), jnp.float32),
        pltpu.VMEM((CHUNK,), jnp.float32),
        pltpu.VMEM((CHUNK,), jnp.float32),
        pltpu.VMEM((CHUNK,), jnp.float32),
        pltpu.VMEM((CHUNK,), jnp.float32),
        pltpu.VMEM((CHUNK,), jnp.float32),
        pltpu.VMEM((K,), jnp.float32),
        pltpu.VMEM((K,), jnp.float32),
        pltpu.VMEM((K,), jnp.float32),
        pltpu.SemaphoreType.DMA,
        pltpu.SemaphoreType.DMA,
        pltpu.SemaphoreType.DMA,
        pltpu.SemaphoreType.DMA,
    ],
)(_sc_body)


def _tc_stats(nuparts_ref, mx_ref, se_ref, nll_ref, nu_ref, elbo_ref, nn_ref):
  nu = jnp.sum(nuparts_ref[...], axis=0, keepdims=True)
  nu_ref[...] = nu
  mx = mx_ref[...]
  se = se_ref[...]
  logz = mx + jnp.log(se)
  elbo = jnp.sum(logz) * (1.0 / N)
  qn = jnp.exp(nll_ref[...] - mx) / se
  nn = jnp.sum(qn)
  elbo_ref[...] = jnp.full((8, 128), elbo, jnp.float32)
  nn_ref[...] = jnp.full((8, 128), nn, jnp.float32)


def _tc_mreduce(mpart_ref, nu_ref, m_ref, acc_ref):
  i = pl.program_id(0)

  @pl.when(i == 0)
  def _():
    acc_ref[...] = jnp.zeros_like(acc_ref)

  acc_ref[...] += mpart_ref[0]

  @pl.when(i == NW - 1)
  def _():
    nub = lax.broadcast_in_dim(nu_ref[0], (D, K), (1,))
    m_ref[...] = (acc_ref[...] / jnp.clip(nub, 1.0, None)).T


def kernel(features, means, log_proportions, candidates):
  feat_flat = features.reshape(-1)
  cand_flat = candidates.reshape(-1)
  meansT_flat = means.T.reshape(-1)

  mparts, nuparts, newcand, mx, se, nll = _sc_kernel(
      feat_flat, cand_flat, meansT_flat, log_proportions)

  nu, elbo, nn = pl.pallas_call(
      _tc_stats,
      out_shape=[
          jax.ShapeDtypeStruct((1, K), jnp.float32),
          jax.ShapeDtypeStruct((8, 128), jnp.float32),
          jax.ShapeDtypeStruct((8, 128), jnp.float32),
      ],
  )(nuparts.reshape(NW, K),
    mx.reshape(N // 128, 128), se.reshape(N // 128, 128),
    nll.reshape(N // 128, 128))

  m = pl.pallas_call(
      _tc_mreduce,
      grid=(NW,),
      in_specs=[
          pl.BlockSpec((1, D, K), lambda i: (i, 0, 0)),
          pl.BlockSpec((1, K), lambda i: (0, 0)),
      ],
      out_specs=pl.BlockSpec((K, D), lambda i: (0, 0)),
      out_shape=jax.ShapeDtypeStruct((K, D), jnp.float32),
      scratch_shapes=[pltpu.VMEM((D, K), jnp.float32)],
  )(mparts.reshape(NW, D, K), nu)

  N_units = nu.reshape(K)
  noise_N = nn[0, 0]
  obs_elbo = elbo[0, 0]
  new_candidates = newcand.reshape(N, C)
  return m, N_units, noise_N, obs_elbo, new_candidates
